# Initial kernel scaffold; baseline (speedup 1.0000x reference)
import jax
import jax.numpy as jnp
from jax.experimental import pallas as pl


def kernel(inp, W1, b1, W2, b2):
    raise NotImplementedError("write your pallas kernel here")


# speedup vs baseline: 6.8662x; 6.8662x over previous
import functools

import jax
import jax.numpy as jnp
from jax.experimental import pallas as pl

TOKENS = 32768
INP = 768
HID = 384
NF = 64
TOP_K = 8
TILE = 1024


def _fused_kernel(x_ref, w1_ref, b1_ref, w2_ref, b2_ref, o_ref):
    x = x_ref[...]
    h = jnp.maximum(
        jnp.dot(x, w1_ref[...], preferred_element_type=jnp.float32)
        + b1_ref[...],
        0.0,
    )
    logits = (
        jnp.dot(h, w2_ref[...], preferred_element_type=jnp.float32)
        + b2_ref[...]
    )
    s = jax.nn.sigmoid(logits)

    col = jax.lax.broadcasted_iota(jnp.int32, s.shape, 1)
    work = s
    mask = jnp.zeros(s.shape, dtype=jnp.bool_)
    for _ in range(TOP_K):
        m = jnp.max(work, axis=1, keepdims=True)
        eq = work == m
        first = jnp.min(jnp.where(eq, col, NF), axis=1, keepdims=True)
        sel = col == first
        mask = jnp.logical_or(mask, sel)
        work = jnp.where(sel, -jnp.inf, work)
    o_ref[...] = jnp.where(mask, s, 0.0)


@functools.partial(jax.jit, static_argnames=())
def kernel(inp, W1, b1, W2, b2):
    b1r = b1.reshape(1, HID)
    b2r = b2.reshape(1, NF)
    grid = (TOKENS // TILE,)
    return pl.pallas_call(
        _fused_kernel,
        grid=grid,
        in_specs=[
            pl.BlockSpec((TILE, INP), lambda i: (i, 0)),
            pl.BlockSpec((INP, HID), lambda i: (0, 0)),
            pl.BlockSpec((1, HID), lambda i: (0, 0)),
            pl.BlockSpec((HID, NF), lambda i: (0, 0)),
            pl.BlockSpec((1, NF), lambda i: (0, 0)),
        ],
        out_specs=pl.BlockSpec((TILE, NF), lambda i: (i, 0)),
        out_shape=jax.ShapeDtypeStruct((TOKENS, NF), jnp.float32),
    )(inp, W1, b1r, W2, b2r)


# speedup vs baseline: 8.9727x; 1.3068x over previous
import functools

import jax
import jax.numpy as jnp
from jax.experimental import pallas as pl

TOKENS = 32768
INP = 768
HID = 384
NF = 64
TOP_K = 8
TILE = 1024


def _fused_kernel(x_ref, w1_ref, b1_ref, w2_ref, b2_ref, o_ref):
    x = x_ref[...]
    h = jnp.maximum(
        jnp.dot(x, w1_ref[...], preferred_element_type=jnp.float32)
        + b1_ref[...],
        0.0,
    )
    logits = (
        jnp.dot(h, w2_ref[...], preferred_element_type=jnp.float32)
        + b2_ref[...]
    )
    s = jax.nn.sigmoid(logits)

    bits = jax.lax.bitcast_convert_type(s, jnp.int32)
    col = jax.lax.broadcasted_iota(jnp.int32, s.shape, 1)
    key = ((bits << 6) | (NF - 1 - col)) ^ jnp.int32(-2147483648)
    knocked = jnp.int32(-2147483648)
    for _ in range(TOP_K):
        m = jnp.max(key, axis=1, keepdims=True)
        key = jnp.where(key == m, knocked, key)
    o_ref[...] = jnp.where(key == knocked, s, 0.0)


@functools.partial(jax.jit, static_argnames=())
def kernel(inp, W1, b1, W2, b2):
    b1r = b1.reshape(1, HID)
    b2r = b2.reshape(1, NF)
    grid = (TOKENS // TILE,)
    return pl.pallas_call(
        _fused_kernel,
        grid=grid,
        in_specs=[
            pl.BlockSpec((TILE, INP), lambda i: (i, 0)),
            pl.BlockSpec((INP, HID), lambda i: (0, 0)),
            pl.BlockSpec((1, HID), lambda i: (0, 0)),
            pl.BlockSpec((HID, NF), lambda i: (0, 0)),
            pl.BlockSpec((1, NF), lambda i: (0, 0)),
        ],
        out_specs=pl.BlockSpec((TILE, NF), lambda i: (i, 0)),
        out_shape=jax.ShapeDtypeStruct((TOKENS, NF), jnp.float32),
    )(inp, W1, b1r, W2, b2r)


# speedup vs baseline: 12.5656x; 1.4004x over previous
import functools

import jax
import jax.numpy as jnp
from jax.experimental import pallas as pl

TOKENS = 32768
INP = 768
HID = 384
NF = 64
TOP_K = 8
TILE = 1024


def _fused_kernel(x_ref, w1_ref, b1_ref, w2_ref, b2_ref, o_ref):
    x = x_ref[...]
    h = jnp.maximum(
        jnp.dot(x, w1_ref[...], preferred_element_type=jnp.float32)
        + b1_ref[...],
        0.0,
    )
    logits = (
        jnp.dot(h, w2_ref[...], preferred_element_type=jnp.float32)
        + b2_ref[...]
    )
    s = jax.nn.sigmoid(logits)

    knocked = jnp.float32(-1.0)
    work = s
    for _ in range(TOP_K):
        m = jnp.max(work, axis=1, keepdims=True)
        work = jnp.where(work == m, knocked, work)
    o_ref[...] = jnp.where(work == knocked, s, 0.0)


@functools.partial(jax.jit, static_argnames=())
def kernel(inp, W1, b1, W2, b2):
    b1r = b1.reshape(1, HID)
    b2r = b2.reshape(1, NF)
    grid = (TOKENS // TILE,)
    return pl.pallas_call(
        _fused_kernel,
        grid=grid,
        in_specs=[
            pl.BlockSpec((TILE, INP), lambda i: (i, 0)),
            pl.BlockSpec((INP, HID), lambda i: (0, 0)),
            pl.BlockSpec((1, HID), lambda i: (0, 0)),
            pl.BlockSpec((HID, NF), lambda i: (0, 0)),
            pl.BlockSpec((1, NF), lambda i: (0, 0)),
        ],
        out_specs=pl.BlockSpec((TILE, NF), lambda i: (i, 0)),
        out_shape=jax.ShapeDtypeStruct((TOKENS, NF), jnp.float32),
    )(inp, W1, b1r, W2, b2r)


# speedup vs baseline: 13.1593x; 1.0472x over previous
import functools

import jax
import jax.numpy as jnp
from jax.experimental import pallas as pl

TOKENS = 32768
INP = 768
HID = 384
NF = 64
TOP_K = 8
TILE = 2048


def _fused_kernel(x_ref, w1_ref, b1_ref, w2_ref, b2_ref, o_ref):
    x = x_ref[...]
    h = jnp.maximum(
        jnp.dot(x, w1_ref[...], preferred_element_type=jnp.float32)
        + b1_ref[...],
        0.0,
    )
    logits = (
        jnp.dot(h, w2_ref[...], preferred_element_type=jnp.float32)
        + b2_ref[...]
    )
    s = jax.nn.sigmoid(logits)

    knocked = jnp.float32(-1.0)
    work = s
    for _ in range(TOP_K):
        m = jnp.max(work, axis=1, keepdims=True)
        work = jnp.where(work == m, knocked, work)
    o_ref[...] = jnp.where(work == knocked, s, 0.0)


@functools.partial(jax.jit, static_argnames=())
def kernel(inp, W1, b1, W2, b2):
    b1r = b1.reshape(1, HID)
    b2r = b2.reshape(1, NF)
    grid = (TOKENS // TILE,)
    return pl.pallas_call(
        _fused_kernel,
        grid=grid,
        in_specs=[
            pl.BlockSpec((TILE, INP), lambda i: (i, 0)),
            pl.BlockSpec((INP, HID), lambda i: (0, 0)),
            pl.BlockSpec((1, HID), lambda i: (0, 0)),
            pl.BlockSpec((HID, NF), lambda i: (0, 0)),
            pl.BlockSpec((1, NF), lambda i: (0, 0)),
        ],
        out_specs=pl.BlockSpec((TILE, NF), lambda i: (i, 0)),
        out_shape=jax.ShapeDtypeStruct((TOKENS, NF), jnp.float32),
    )(inp, W1, b1r, W2, b2r)
